# Initial kernel scaffold; baseline (speedup 1.0000x reference)
import jax
import jax.numpy as jnp
from jax.experimental import pallas as pl


def kernel(item_fea, W_stars, W_postalcode):
    raise NotImplementedError("write your pallas kernel here")


# speedup vs baseline: 2.3419x; 2.3419x over previous
import dataclasses
import functools

import jax
import jax.numpy as jnp
from jax import lax
from jax.experimental import pallas as pl
from jax.experimental.pallas import tpu as pltpu
from jax.experimental.pallas import tpu_sc as plsc

_NUM_STARS = 10
_NUM_POSTAL = 1000
_D = 128
_B = 16384

_NC = 2
_NS = 16
_L = 16
_NW = _NC * _NS
_ITEMS_PER_W = _B // _NW
_ROWS_PER_W = 2 * _ITEMS_PER_W
_CHUNK = 128
_N_CHUNKS = _ROWS_PER_W // _CHUNK


_CP = pltpu.CompilerParams()
if "needs_layout_passes" in pltpu.CompilerParams.__dataclass_fields__:
    _CP = dataclasses.replace(_CP, needs_layout_passes=False)


@functools.partial(
    pl.kernel,
    mesh=plsc.VectorSubcoreMesh(core_axis_name="c", subcore_axis_name="s"),
    compiler_params=_CP,
    out_type=jax.ShapeDtypeStruct((2 * _B, _D), jnp.float32),
    scratch_types=[
        pltpu.VMEM((_ITEMS_PER_W, 3), jnp.int32),
        pltpu.VMEM((_ROWS_PER_W,), jnp.int32),
        pltpu.VMEM((2, _CHUNK, _D), jnp.float32),
        pltpu.VMEM_SHARED((_NUM_STARS + _NUM_POSTAL, _D), jnp.float32),
        pltpu.SemaphoreType.DMA,
        pltpu.SemaphoreType.DMA,
        pltpu.SemaphoreType.DMA,
    ],
)
def _emb_lookup(fea_hbm, stars_hbm, postal_hbm, out_hbm,
                fea_v, idx_v, buf_v, table_sh, gsem, osem0, osem1):
    cid = lax.axis_index("c")
    sid = lax.axis_index("s")
    wid = sid * _NC + cid

    @pl.when(sid == 0)
    def _():
        pltpu.sync_copy(stars_hbm, table_sh.at[pl.ds(0, _NUM_STARS)])
        pltpu.sync_copy(postal_hbm, table_sh.at[pl.ds(_NUM_STARS, _NUM_POSTAL)])
    plsc.subcore_barrier()

    pltpu.sync_copy(fea_hbm.at[pl.ds(wid * _ITEMS_PER_W, _ITEMS_PER_W)], fea_v)

    col1 = jnp.full((_L,), 1, jnp.int32)
    col2 = jnp.full((_L,), 2, jnp.int32)
    lane = lax.iota(jnp.int32, _L)

    @pl.loop(0, _ITEMS_PER_W // _L)
    def _(i):
        r = i * _L + lane
        stars = plsc.load_gather(fea_v, [r, col1])
        postal = plsc.load_gather(fea_v, [r, col2])
        plsc.store_scatter(idx_v, [2 * r], stars)
        plsc.store_scatter(idx_v, [2 * r + 1], postal + _NUM_STARS)

    base_row = wid * _ROWS_PER_W
    osem = (osem0, osem1)
    gh = {}
    sh = {}

    def start_gather(c, slot):
        return pltpu.async_copy(
            table_sh.at[idx_v.at[pl.ds(c * _CHUNK, _CHUNK)]], buf_v.at[slot], gsem)

    gh[0] = start_gather(0, 0)
    for c in range(_N_CHUNKS):
        s = c % 2
        gh[c].wait()
        if c + 1 < _N_CHUNKS:
            if c >= 1:
                sh[c - 1].wait()
            gh[c + 1] = start_gather(c + 1, 1 - s)
        sh[c] = pltpu.async_copy(
            buf_v.at[s], out_hbm.at[pl.ds(base_row + c * _CHUNK, _CHUNK)], osem[s])
    sh[_N_CHUNKS - 2].wait()
    sh[_N_CHUNKS - 1].wait()


def kernel(item_fea, W_stars, W_postalcode):
    fea = item_fea.astype(jnp.int32)
    out = _emb_lookup(fea, W_stars, W_postalcode)
    return out.reshape(_B, 2 * _D)


# speedup vs baseline: 3.5843x; 1.5305x over previous
import dataclasses
import functools

import jax
import jax.numpy as jnp
from jax import lax
from jax.experimental import pallas as pl
from jax.experimental.pallas import tpu as pltpu
from jax.experimental.pallas import tpu_sc as plsc

_NUM_STARS = 10
_NUM_POSTAL = 1000
_D = 128
_B = 16384

_NC = 2
_NS = 16
_L = 16
_NW = _NC * _NS
_ITEMS_PER_W = _B // _NW
_ROWS_PER_W = 2 * _ITEMS_PER_W
_CHUNK = 128
_N_CHUNKS = _ROWS_PER_W // _CHUNK


_CP = pltpu.CompilerParams()
if "needs_layout_passes" in pltpu.CompilerParams.__dataclass_fields__:
    _CP = dataclasses.replace(_CP, needs_layout_passes=False)


@functools.partial(
    pl.kernel,
    mesh=plsc.VectorSubcoreMesh(core_axis_name="c", subcore_axis_name="s"),
    compiler_params=_CP,
    out_type=jax.ShapeDtypeStruct((_B, 2 * _D), jnp.float32),
    scratch_types=[
        pltpu.VMEM((_ITEMS_PER_W, 3), jnp.int32),
        pltpu.VMEM((_ROWS_PER_W,), jnp.int32),
        pltpu.VMEM((2, _CHUNK, _D), jnp.float32),
        pltpu.VMEM_SHARED((_NUM_STARS + _NUM_POSTAL, _D), jnp.float32),
        pltpu.SemaphoreType.DMA,
        pltpu.SemaphoreType.DMA,
        pltpu.SemaphoreType.DMA,
    ],
)
def _emb_lookup(fea_hbm, stars_hbm, postal_hbm, out_hbm,
                fea_v, idx_v, buf_v, table_sh, gsem, osem0, osem1):
    cid = lax.axis_index("c")
    sid = lax.axis_index("s")
    wid = sid * _NC + cid

    @pl.when(sid == 0)
    def _():
        pltpu.sync_copy(stars_hbm, table_sh.at[pl.ds(0, _NUM_STARS)])
        pltpu.sync_copy(postal_hbm, table_sh.at[pl.ds(_NUM_STARS, _NUM_POSTAL)])
    plsc.subcore_barrier()

    pltpu.sync_copy(fea_hbm.at[pl.ds(wid * _ITEMS_PER_W, _ITEMS_PER_W)], fea_v)

    col1 = jnp.full((_L,), 1, jnp.int32)
    col2 = jnp.full((_L,), 2, jnp.int32)
    lane = lax.iota(jnp.int32, _L)

    @pl.loop(0, _ITEMS_PER_W // _L)
    def _(i):
        r = i * _L + lane
        stars = plsc.load_gather(fea_v, [r, col1])
        postal = plsc.load_gather(fea_v, [r, col2])
        plsc.store_scatter(idx_v, [2 * r], stars)
        plsc.store_scatter(idx_v, [2 * r + 1], postal + _NUM_STARS)

    base_item = wid * _ITEMS_PER_W
    osem = (osem0, osem1)
    gh = {}
    sh = {}

    def start_gather(c, slot):
        return pltpu.async_copy(
            table_sh.at[idx_v.at[pl.ds(c * _CHUNK, _CHUNK)]], buf_v.at[slot], gsem)

    gh[0] = start_gather(0, 0)
    for c in range(_N_CHUNKS):
        s = c % 2
        gh[c].wait()
        if c + 1 < _N_CHUNKS:
            if c >= 1:
                sh[c - 1].wait()
            gh[c + 1] = start_gather(c + 1, 1 - s)
        sh[c] = pltpu.async_copy(
            buf_v.at[s].reshape(_CHUNK // 2, 2 * _D),
            out_hbm.at[pl.ds(base_item + c * (_CHUNK // 2), _CHUNK // 2)],
            osem[s])
    sh[_N_CHUNKS - 2].wait()
    sh[_N_CHUNKS - 1].wait()


def kernel(item_fea, W_stars, W_postalcode):
    fea = item_fea.astype(jnp.int32)
    return _emb_lookup(fea, W_stars, W_postalcode)


# speedup vs baseline: 3.6685x; 1.0235x over previous
import dataclasses
import functools

import jax
import jax.numpy as jnp
from jax import lax
from jax.experimental import pallas as pl
from jax.experimental.pallas import tpu as pltpu
from jax.experimental.pallas import tpu_sc as plsc

_NUM_STARS = 10
_NUM_POSTAL = 1000
_D = 128
_B = 16384

_NC = 2
_NS = 16
_L = 16
_NW = _NC * _NS
_ITEMS_PER_W = _B // _NW
_ROWS_PER_W = 2 * _ITEMS_PER_W
_CHUNK = 128
_N_CHUNKS = _ROWS_PER_W // _CHUNK
_NBUF = 3


_CP = pltpu.CompilerParams()
if "needs_layout_passes" in pltpu.CompilerParams.__dataclass_fields__:
    _CP = dataclasses.replace(_CP, needs_layout_passes=False)


@functools.partial(
    pl.kernel,
    mesh=plsc.VectorSubcoreMesh(core_axis_name="c", subcore_axis_name="s"),
    compiler_params=_CP,
    out_type=jax.ShapeDtypeStruct((_B, 2 * _D), jnp.float32),
    scratch_types=[
        pltpu.VMEM((_ITEMS_PER_W, 3), jnp.int32),
        pltpu.VMEM((_ROWS_PER_W,), jnp.int32),
        pltpu.VMEM((_NBUF, _CHUNK, _D), jnp.float32),
        pltpu.VMEM_SHARED((_NUM_STARS + _NUM_POSTAL, _D), jnp.float32),
        pltpu.SemaphoreType.DMA,
        pltpu.SemaphoreType.DMA,
        pltpu.SemaphoreType.DMA,
    ],
)
def _emb_lookup(fea_hbm, stars_hbm, postal_hbm, out_hbm,
                fea_v, idx_v, buf_v, table_sh, sem0, sem1, sem2):
    cid = lax.axis_index("c")
    sid = lax.axis_index("s")
    wid = sid * _NC + cid

    @pl.when(sid < 5)
    def _():
        pltpu.sync_copy(postal_hbm.at[pl.ds(sid * 200, 200)],
                        table_sh.at[pl.ds(sid * 200, 200)])
    @pl.when(sid == 5)
    def _():
        pltpu.sync_copy(stars_hbm, table_sh.at[pl.ds(_NUM_POSTAL, _NUM_STARS)])
    plsc.subcore_barrier()

    pltpu.sync_copy(fea_hbm.at[pl.ds(wid * _ITEMS_PER_W, _ITEMS_PER_W)], fea_v)

    col1 = jnp.full((_L,), 1, jnp.int32)
    col2 = jnp.full((_L,), 2, jnp.int32)
    lane = lax.iota(jnp.int32, _L)

    @pl.loop(0, _ITEMS_PER_W // _L)
    def _(i):
        r = i * _L + lane
        stars = plsc.load_gather(fea_v, [r, col1])
        postal = plsc.load_gather(fea_v, [r, col2])
        plsc.store_scatter(idx_v, [2 * r], stars + _NUM_POSTAL)
        plsc.store_scatter(idx_v, [2 * r + 1], postal)

    base_item = wid * _ITEMS_PER_W
    sems = (sem0, sem1, sem2)
    gh = {}
    sh = {}

    def start_gather(c):
        b = c % _NBUF
        return pltpu.async_copy(
            table_sh.at[idx_v.at[pl.ds(c * _CHUNK, _CHUNK)]], buf_v.at[b], sems[b])

    def start_store(c):
        b = c % _NBUF
        return pltpu.async_copy(
            buf_v.at[b].reshape(_CHUNK // 2, 2 * _D),
            out_hbm.at[pl.ds(base_item + c * (_CHUNK // 2), _CHUNK // 2)],
            sems[b])

    for c in range(_NBUF):
        gh[c] = start_gather(c)
    for c in range(_N_CHUNKS):
        gh[c].wait()
        sh[c] = start_store(c)
        if c + _NBUF < _N_CHUNKS:
            sh[c].wait()
            gh[c + _NBUF] = start_gather(c + _NBUF)
    for c in range(_N_CHUNKS - _NBUF, _N_CHUNKS):
        sh[c].wait()


def kernel(item_fea, W_stars, W_postalcode):
    fea = item_fea.astype(jnp.int32)
    return _emb_lookup(fea, W_stars, W_postalcode)
